# baseline (speedup 1.0000x reference)
import functools

import jax
import jax.numpy as jnp
from jax import lax
from jax.experimental import pallas as pl
from jax.experimental.pallas import tpu as pltpu

BB, TT, DD = 2, 2048, 1024
EE = 2
FF = 4096
NN = BB * TT
TILE = 256
NPAD = NN + TILE
NB = NPAD // TILE
NBS = NN // TILE
FT = 512
NF = FF // FT


def _router_body(x_ref, wr_ref, b_ref, idx_ref, xb_ref):
    xblk = x_ref[...]
    logits = lax.dot_general(xblk, wr_ref[...], (((1,), (1,)), ((), ())),
                             preferred_element_type=jnp.float32)
    logits = logits + b_ref[...]
    a = logits[:, 0:1]
    b = logits[:, 1:2]
    idx_ref[...] = jnp.where(a >= b, 0, 1).astype(jnp.int32)
    xb_ref[...] = xblk.astype(jnp.bfloat16)


def _router(flat, Wr, router_bias):
    return pl.pallas_call(
        _router_body,
        grid=(1,),
        in_specs=[
            pl.BlockSpec((NN, DD), lambda i: (0, 0)),
            pl.BlockSpec((EE, DD), lambda i: (0, 0)),
            pl.BlockSpec((1, EE), lambda i: (0, 0)),
        ],
        out_specs=[
            pl.BlockSpec((NN, 1), lambda i: (0, 0)),
            pl.BlockSpec((NN, DD), lambda i: (0, 0)),
        ],
        out_shape=[
            jax.ShapeDtypeStruct((NN, 1), jnp.int32),
            jax.ShapeDtypeStruct((NN, DD), jnp.bfloat16),
        ],
    )(flat, Wr, router_bias.reshape(1, EE))


def _ffn_body(be_ref, xs_ref, wg_ref, wu_ref, wd_ref, out_ref, *, tile):
    f = pl.program_id(0)
    t = pl.program_id(1)
    sl = pl.ds(t * tile, tile)
    xblk = xs_ref[sl, :]
    g = wg_ref[0].astype(jnp.bfloat16)
    u = wu_ref[0].astype(jnp.bfloat16)
    d = wd_ref[0].astype(jnp.bfloat16)
    gg = lax.dot_general(xblk, g, (((1,), (1,)), ((), ())),
                         preferred_element_type=jnp.float32)
    uu = lax.dot_general(xblk, u, (((1,), (1,)), ((), ())),
                         preferred_element_type=jnp.float32)
    h = (gg * jax.nn.sigmoid(gg) * uu).astype(jnp.bfloat16)
    y = lax.dot_general(h, d, (((1,), (1,)), ((), ())),
                        preferred_element_type=jnp.float32)

    @pl.when(f == 0)
    def _():
        out_ref[sl, :] = y

    @pl.when(f != 0)
    def _():
        out_ref[sl, :] = out_ref[sl, :] + y


def _grouped_ffn(be, xs, Wg, Wu, Wd, n_rows, n_blocks, tile):
    grid_spec = pltpu.PrefetchScalarGridSpec(
        num_scalar_prefetch=1,
        grid=(NF, n_blocks),
        in_specs=[
            pl.BlockSpec((n_rows, DD), lambda f, t, be: (0, 0)),
            pl.BlockSpec((1, FT, DD), lambda f, t, be: (be[t], f, 0)),
            pl.BlockSpec((1, FT, DD), lambda f, t, be: (be[t], f, 0)),
            pl.BlockSpec((1, DD, FT), lambda f, t, be: (be[t], 0, f)),
        ],
        out_specs=pl.BlockSpec((n_rows, DD), lambda f, t, be: (0, 0)),
    )
    return pl.pallas_call(
        functools.partial(_ffn_body, tile=tile),
        grid_spec=grid_spec,
        out_shape=jax.ShapeDtypeStruct((n_rows, DD), jnp.float32),
    )(be, xs, Wg, Wu, Wd)


def _add_body(a_ref, b_ref, o_ref):
    o_ref[...] = a_ref[...] + b_ref[...]


def _add(a, b):
    return pl.pallas_call(
        _add_body,
        grid=(NBS,),
        in_specs=[
            pl.BlockSpec((TILE, DD), lambda i: (i, 0)),
            pl.BlockSpec((TILE, DD), lambda i: (i, 0)),
        ],
        out_specs=pl.BlockSpec((TILE, DD), lambda i: (i, 0)),
        out_shape=jax.ShapeDtypeStruct((NN, DD), jnp.float32),
    )(a, b)


def kernel(x, Wr, router_bias, Wg, Wu, Wd, Sg, Su, Sd):
    flat = x.reshape(NN, DD)
    idx2d, xb = _router(flat, Wr, router_bias)
    idx = idx2d.reshape(NN)

    m0 = idx == 0
    c0 = jnp.cumsum(m0.astype(jnp.int32))
    c1 = jnp.cumsum((~m0).astype(jnp.int32))
    pos = jnp.where(m0, c0 - 1, NPAD - c1)
    src = jnp.zeros((NPAD,), jnp.int32).at[pos].set(
        jnp.arange(NN, dtype=jnp.int32))
    cnt0 = c0[-1]
    g0 = (cnt0 + TILE - 1) // TILE
    be = (jnp.arange(NB, dtype=jnp.int32) >= g0).astype(jnp.int32)

    xs = jnp.take(xb, src, axis=0)

    ys = _grouped_ffn(be, xs, Wg, Wu, Wd, NPAD, NB, TILE)
    sh = _grouped_ffn(jnp.zeros((NBS,), jnp.int32), xb,
                      Sg[None], Su[None], Sd[None], NN, NBS, TILE)

    tmp = jnp.take(ys, pos, axis=0)

    out = _add(tmp, sh)
    return out.reshape(BB, TT, DD)
